# baseline (speedup 1.0000x reference)
import functools

import jax
import jax.numpy as jnp
from jax import lax
from jax.experimental import pallas as pl
from jax.experimental.pallas import tpu as pltpu
from jax.experimental.pallas import tpu_sc as plsc

N = 10000
E = 160000
D = 256
HALF = 128
NC = 2
NS = 16
L = 16
NP = 10240
NPS = NP // NS
CH = 80
EPS = E // NS
NECH = EPS // CH
NNCH = NPS // CH
RB = 512


def _lin_body(xa_ref, xb_ref, w_ref, b_ref, o_ref):
    x = jnp.concatenate([xa_ref[...].reshape(RB, HALF),
                         xb_ref[...].reshape(RB, HALF)], axis=1)
    w = w_ref[0]
    o = lax.dot_general(x, w, (((1,), (1,)), ((), ())),
                        preferred_element_type=jnp.float32)
    o_ref[0] = o + b_ref[0]


def _linear_halves_from_full(x, W, b):
    w3 = W.reshape(NC, HALF, D)
    b2 = b.reshape(NC, 1, HALF)
    grid = (NC, NP // RB)
    return pl.pallas_call(
        _lin_body,
        grid=grid,
        in_specs=[
            pl.BlockSpec((RB, HALF), lambda c, i: (i, 0)),
            pl.BlockSpec((RB, HALF), lambda c, i: (i, 1)),
            pl.BlockSpec((1, HALF, D), lambda c, i: (c, 0, 0)),
            pl.BlockSpec((1, 1, HALF), lambda c, i: (c, 0, 0)),
        ],
        out_specs=pl.BlockSpec((1, RB, HALF), lambda c, i: (c, i, 0)),
        out_shape=jax.ShapeDtypeStruct((NC, NP, HALF), jnp.float32),
    )(x, x, w3, b2)


def _linear_halves_from_halves(h, W, b):
    w3 = W.reshape(NC, HALF, D)
    b2 = b.reshape(NC, 1, HALF)
    grid = (NC, NP // RB)
    return pl.pallas_call(
        _lin_body,
        grid=grid,
        in_specs=[
            pl.BlockSpec((1, RB, HALF), lambda c, i: (0, i, 0)),
            pl.BlockSpec((1, RB, HALF), lambda c, i: (1, i, 0)),
            pl.BlockSpec((1, HALF, D), lambda c, i: (c, 0, 0)),
            pl.BlockSpec((1, 1, HALF), lambda c, i: (c, 0, 0)),
        ],
        out_specs=pl.BlockSpec((1, RB, HALF), lambda c, i: (c, i, 0)),
        out_shape=jax.ShapeDtypeStruct((NC, NP, HALF), jnp.float32),
    )(h, h, w3, b2)


def _rsqrt16(x):
    i = lax.bitcast_convert_type(x, jnp.int32)
    magic = jnp.full((L,), 0x5F3759DF, jnp.int32)
    y = lax.bitcast_convert_type(magic - lax.shift_right_logical(i, 1),
                                 jnp.float32)
    half = x * (-0.5)
    for _ in range(3):
        y = y * (half * y * y + 1.5)
    return y


def _splat(ref, i):
    return plsc.load_gather(ref, [jnp.full((L,), i, jnp.int32)])


def _sc_degree():
    mesh = plsc.VectorSubcoreMesh(core_axis_name="c", subcore_axis_name="s",
                                  num_cores=NC)
    scratch = dict(
        epk_v=[pltpu.VMEM((3, CH), jnp.int32)] * 3,
        dcol_v=[pltpu.VMEM((CH,), jnp.int32)] * 3,
        t_v=pltpu.VMEM((NPS,), jnp.float32),
        ones_v=pltpu.VMEM((CH,), jnp.float32),
        deg_sh=pltpu.VMEM_SHARED((NP,), jnp.float32),
        sem_e=pltpu.SemaphoreType.DMA,
        sem_d=[pltpu.SemaphoreType.DMA] * 3,
    )

    @functools.partial(
        pl.kernel, out_type=jax.ShapeDtypeStruct((NP,), jnp.float32),
        mesh=mesh, scratch_types=scratch,
        compiler_params=pltpu.CompilerParams(needs_layout_passes=False))
    def kdeg(epk_hbm, dis_out, *, epk_v, dcol_v, t_v, ones_v, deg_sh,
             sem_e, sem_d):
        c = lax.axis_index("c")
        s = lax.axis_index("s")
        nbase = s * NPS
        zero16 = jnp.zeros((L,), jnp.float32)
        one16 = jnp.ones((L,), jnp.float32)

        def fill(ref, n, vec):
            def st(i, _):
                ref[pl.ds(i * L, L)] = vec
                return 0
            lax.fori_loop(0, n // L, st, 0)

        fill(ones_v, CH, one16)
        fill(t_v, NPS, zero16)
        pltpu.sync_copy(t_v, deg_sh.at[pl.ds(nbase, NPS)])
        plsc.subcore_barrier()

        def prep_d(b):
            @plsc.parallel_loop(0, CH // L, unroll=5)
            def grp(g):
                sl = pl.ds(g * L, L)
                dcol_v[b][sl] = epk_v[b][1, sl]

        def issue_dscat(b):
            pltpu.async_copy(ones_v, deg_sh.at[dcol_v[b]], sem_d[b],
                             add=True)

        def wait_dscat(b):
            pltpu.make_async_copy(ones_v, deg_sh.at[dcol_v[b]],
                                  sem_d[b]).wait()

        def deg_iter(j, b, first):
            n = (b + 1) % 3
            p = (b + 2) % 3
            pltpu.make_async_copy(epk_hbm.at[s, j + 1], epk_v[n],
                                  sem_e).wait()
            prep_d(n)
            pltpu.async_copy(epk_hbm.at[s, j + 2], epk_v[p], sem_e)
            if not first:
                wait_dscat(p)
            issue_dscat(b)

        pltpu.sync_copy(epk_hbm.at[s, 0], epk_v[0])
        prep_d(0)
        pltpu.async_copy(epk_hbm.at[s, 1], epk_v[1], sem_e)
        deg_iter(0, 0, True)
        deg_iter(1, 1, False)

        def deg_triple(t, _):
            deg_iter(3 * t + 2, 2, False)
            deg_iter(3 * t + 3, 0, False)
            deg_iter(3 * t + 4, 1, False)
            return 0
        lax.fori_loop(0, (NECH - 2) // 3, deg_triple, 0)
        wait_dscat(1)
        pltpu.make_async_copy(epk_hbm.at[s, NECH + 1],
                              epk_v[(NECH + 1) % 3], sem_e).wait()
        plsc.subcore_barrier()

        pltpu.sync_copy(deg_sh.at[pl.ds(nbase, NPS)], t_v)

        def mk_dis(i, _):
            sl = pl.ds(i * L, L)
            t_v[sl] = _rsqrt16(t_v[sl] + 1.0)
            return 0
        lax.fori_loop(0, NPS // L, mk_dis, 0)

        @pl.when(c == 0)
        def _():
            pltpu.sync_copy(t_v, dis_out.at[pl.ds(nbase, NPS)])

    return kdeg


def _make_sc_propagate(final: bool):
    mesh = plsc.VectorSubcoreMesh(core_axis_name="c", subcore_axis_name="s",
                                  num_cores=NC)
    if final:
        out_type = jax.ShapeDtypeStruct((N, D), jnp.float32)
    else:
        out_type = jax.ShapeDtypeStruct((NC * NP, HALF), jnp.float32)

    scratch = dict(
        epk_v=[pltpu.VMEM((3, CH), jnp.int32)] * 3,
        rowo_v=[pltpu.VMEM((CH,), jnp.int32)] * 3,
        col_v=[pltpu.VMEM((CH,), jnp.int32)] * 3,
        w_v=[pltpu.VMEM((CH,), jnp.float32)] * 3,
        rows_v=[pltpu.VMEM((CH, HALF), jnp.float32)] * 3,
        dis_v=pltpu.VMEM((NP,), jnp.float32),
        acc_sh=pltpu.VMEM_SHARED((NP, HALF), jnp.float32),
        sem_e=pltpu.SemaphoreType.DMA,
        sem_g=[pltpu.SemaphoreType.DMA] * 3,
        sem_s=[pltpu.SemaphoreType.DMA] * 3,
    )

    @functools.partial(
        pl.kernel, out_type=out_type, mesh=mesh, scratch_types=scratch,
        compiler_params=pltpu.CompilerParams(needs_layout_passes=False))
    def kprop(epk_hbm, hw_hbm, dis_hbm, h_out, *, epk_v, rowo_v, col_v,
              w_v, rows_v, dis_v, acc_sh, sem_e, sem_g, sem_s):
        c = lax.axis_index("c")
        s = lax.axis_index("s")
        nbase = s * NPS
        coff = jnp.full((L,), c * NP, jnp.int32)

        pltpu.sync_copy(dis_hbm, dis_v)

        def init_chunk(k, _):
            base = nbase + k * CH
            buf = rows_v[0]
            pltpu.sync_copy(hw_hbm.at[pl.ds(c * NP + base, CH)], buf)

            @plsc.parallel_loop(0, CH, unroll=4)
            def init_e(e):
                dvec = _splat(dis_v, base + e)
                for g in range(HALF // L):
                    sl = pl.ds(g * L, L)
                    buf[e, sl] = buf[e, sl] * dvec
            pltpu.sync_copy(buf, acc_sh.at[pl.ds(base, CH)])
            return 0
        lax.fori_loop(0, NNCH, init_chunk, 0)
        plsc.subcore_barrier()

        def prep(b):
            @plsc.parallel_loop(0, CH // L, unroll=5)
            def grp(g):
                sl = pl.ds(g * L, L)
                r16 = epk_v[b][0, sl]
                rowo_v[b][sl] = r16 + coff
                d16 = plsc.load_gather(dis_v, [r16])
                e16 = lax.bitcast_convert_type(epk_v[b][2, sl], jnp.float32)
                w_v[b][sl] = d16 * e16
                col_v[b][sl] = epk_v[b][1, sl]

        def issue_gather(b):
            pltpu.async_copy(hw_hbm.at[rowo_v[b]], rows_v[b], sem_g[b])

        def wait_gather(b):
            pltpu.make_async_copy(hw_hbm.at[rowo_v[b]], rows_v[b],
                                  sem_g[b]).wait()

        def issue_scatter(b):
            pltpu.async_copy(rows_v[b], acc_sh.at[col_v[b]], sem_s[b],
                             add=True)

        def wait_scatter(b):
            pltpu.make_async_copy(rows_v[b], acc_sh.at[col_v[b]],
                                  sem_s[b]).wait()

        def scale(b):
            @plsc.parallel_loop(0, CH, unroll=8)
            def scale_e(e):
                wvec = _splat(w_v[b], e)
                for g in range(HALF // L):
                    sl = pl.ds(g * L, L)
                    rows_v[b][e, sl] = rows_v[b][e, sl] * wvec

        def edge_iter(j, b, first):
            n = (b + 1) % 3
            p = (b + 2) % 3
            pltpu.make_async_copy(epk_hbm.at[s, j + 1], epk_v[n],
                                  sem_e).wait()
            prep(n)
            issue_gather(n)
            pltpu.async_copy(epk_hbm.at[s, j + 2], epk_v[p], sem_e)
            wait_gather(b)
            scale(b)
            if not first:
                wait_scatter(p)
            issue_scatter(b)

        pltpu.sync_copy(epk_hbm.at[s, 0], epk_v[0])
        prep(0)
        issue_gather(0)
        pltpu.async_copy(epk_hbm.at[s, 1], epk_v[1], sem_e)
        edge_iter(0, 0, True)
        edge_iter(1, 1, False)

        def edge_triple(t, _):
            edge_iter(3 * t + 2, 2, False)
            edge_iter(3 * t + 3, 0, False)
            edge_iter(3 * t + 4, 1, False)
            return 0
        lax.fori_loop(0, (NECH - 2) // 3, edge_triple, 0)
        wait_scatter(1)
        wait_gather(2)
        pltpu.make_async_copy(epk_hbm.at[s, NECH + 1],
                              epk_v[(NECH + 1) % 3], sem_e).wait()
        plsc.subcore_barrier()

        def flush_chunk(k, _):
            base = nbase + k * CH
            buf = rows_v[0]

            def do_flush():
                pltpu.sync_copy(acc_sh.at[pl.ds(base, CH)], buf)

                @plsc.parallel_loop(0, CH, unroll=4)
                def flush_e(e):
                    dvec = _splat(dis_v, base + e)
                    for g in range(HALF // L):
                        sl = pl.ds(g * L, L)
                        v = buf[e, sl] * dvec
                        if final:
                            v = 1.0 / (1.0 + jnp.exp(-v))
                        else:
                            v = jnp.maximum(v, 0.0)
                        buf[e, sl] = v
                if final:
                    pltpu.sync_copy(
                        buf, h_out.at[pl.ds(base, CH),
                                      pl.ds(c * HALF, HALF)])
                else:
                    pltpu.sync_copy(
                        buf, h_out.at[pl.ds(c * NP + base, CH)])

            if final:
                @pl.when(base < N)
                def _():
                    do_flush()
            else:
                do_flush()
            return 0
        lax.fori_loop(0, NNCH, flush_chunk, 0)

    return kprop


_sc_deg = _sc_degree()
_sc_prop_mid = _make_sc_propagate(False)
_sc_prop_final = _make_sc_propagate(True)


@jax.jit
def kernel(x, edge_index, edge_weight, W1, b1, W2, b2):
    row = edge_index[0].reshape(NS, NECH, CH)
    col = edge_index[1].reshape(NS, NECH, CH)
    ewb = lax.bitcast_convert_type(
        edge_weight.reshape(NS, NECH, CH), jnp.int32)
    epk = jnp.stack([row, col, ewb], axis=2)
    epk = jnp.pad(epk, ((0, 0), (0, 2), (0, 0), (0, 0)))

    dis = _sc_deg(epk)
    xp = jnp.pad(x, ((0, NP - N), (0, 0)))
    hw1 = _linear_halves_from_full(xp, W1, b1).reshape(NC * NP, HALF)
    h1 = _sc_prop_mid(epk, hw1, dis).reshape(NC, NP, HALF)
    hw2 = _linear_halves_from_halves(h1, W2, b2).reshape(NC * NP, HALF)
    return _sc_prop_final(epk, hw2, dis)


# speedup vs baseline: 1.0493x; 1.0493x over previous
import functools

import jax
import jax.numpy as jnp
from jax import lax
from jax.experimental import pallas as pl
from jax.experimental.pallas import tpu as pltpu
from jax.experimental.pallas import tpu_sc as plsc

N = 10000
E = 160000
D = 256
HALF = 128
NC = 2
NS = 16
L = 16
NP = 10240
NPS = NP // NS
CH = 80
EPS = E // NS
NECH = EPS // CH
NNCH = NPS // CH
RB = 512


def _lin_body(xa_ref, xb_ref, w_ref, b_ref, o_ref):
    x = jnp.concatenate([xa_ref[...].reshape(RB, HALF),
                         xb_ref[...].reshape(RB, HALF)], axis=1)
    w = w_ref[0]
    o = lax.dot_general(x, w, (((1,), (1,)), ((), ())),
                        preferred_element_type=jnp.float32)
    o_ref[0] = o + b_ref[0]


def _linear_halves_from_full(x, W, b):
    w3 = W.reshape(NC, HALF, D)
    b2 = b.reshape(NC, 1, HALF)
    grid = (NC, NP // RB)
    return pl.pallas_call(
        _lin_body,
        grid=grid,
        in_specs=[
            pl.BlockSpec((RB, HALF), lambda c, i: (i, 0)),
            pl.BlockSpec((RB, HALF), lambda c, i: (i, 1)),
            pl.BlockSpec((1, HALF, D), lambda c, i: (c, 0, 0)),
            pl.BlockSpec((1, 1, HALF), lambda c, i: (c, 0, 0)),
        ],
        out_specs=pl.BlockSpec((1, RB, HALF), lambda c, i: (c, i, 0)),
        out_shape=jax.ShapeDtypeStruct((NC, NP, HALF), jnp.float32),
    )(x, x, w3, b2)


def _linear_halves_from_halves(h, W, b):
    w3 = W.reshape(NC, HALF, D)
    b2 = b.reshape(NC, 1, HALF)
    grid = (NC, NP // RB)
    return pl.pallas_call(
        _lin_body,
        grid=grid,
        in_specs=[
            pl.BlockSpec((1, RB, HALF), lambda c, i: (0, i, 0)),
            pl.BlockSpec((1, RB, HALF), lambda c, i: (1, i, 0)),
            pl.BlockSpec((1, HALF, D), lambda c, i: (c, 0, 0)),
            pl.BlockSpec((1, 1, HALF), lambda c, i: (c, 0, 0)),
        ],
        out_specs=pl.BlockSpec((1, RB, HALF), lambda c, i: (c, i, 0)),
        out_shape=jax.ShapeDtypeStruct((NC, NP, HALF), jnp.float32),
    )(h, h, w3, b2)


def _rsqrt16(x):
    i = lax.bitcast_convert_type(x, jnp.int32)
    magic = jnp.full((L,), 0x5F3759DF, jnp.int32)
    y = lax.bitcast_convert_type(magic - lax.shift_right_logical(i, 1),
                                 jnp.float32)
    half = x * (-0.5)
    for _ in range(3):
        y = y * (half * y * y + 1.5)
    return y


def _splat(ref, i):
    return plsc.load_gather(ref, [jnp.full((L,), i, jnp.int32)])


def _sc_degree():
    mesh = plsc.VectorSubcoreMesh(core_axis_name="c", subcore_axis_name="s",
                                  num_cores=NC)
    NCC = (NECH + 1) // NC
    scratch = dict(
        epk_v=[pltpu.VMEM((3, CH), jnp.int32)] * 3,
        dcol_v=[pltpu.VMEM((CH,), jnp.int32)] * 3,
        t_v=pltpu.VMEM((NPS,), jnp.float32),
        ones_v=pltpu.VMEM((CH,), jnp.float32),
        deg_sh=pltpu.VMEM_SHARED((NP,), jnp.float32),
        sem_e=pltpu.SemaphoreType.DMA,
        sem_d=[pltpu.SemaphoreType.DMA] * 3,
    )

    @functools.partial(
        pl.kernel, out_type=jax.ShapeDtypeStruct((NC, NP), jnp.float32),
        mesh=mesh, scratch_types=scratch,
        compiler_params=pltpu.CompilerParams(needs_layout_passes=False))
    def kdeg(epk_hbm, pdeg_out, *, epk_v, dcol_v, t_v, ones_v, deg_sh,
             sem_e, sem_d):
        c = lax.axis_index("c")
        s = lax.axis_index("s")
        nbase = s * NPS
        base = c * NCC
        zero16 = jnp.zeros((L,), jnp.float32)
        one16 = jnp.ones((L,), jnp.float32)

        def fill(ref, n, vec):
            def st(i, _):
                ref[pl.ds(i * L, L)] = vec
                return 0
            lax.fori_loop(0, n // L, st, 0)

        fill(ones_v, CH, one16)
        fill(t_v, NPS, zero16)
        pltpu.sync_copy(t_v, deg_sh.at[pl.ds(nbase, NPS)])
        plsc.subcore_barrier()

        def prep_d(b):
            @plsc.parallel_loop(0, CH // L, unroll=5)
            def grp(g):
                sl = pl.ds(g * L, L)
                dcol_v[b][sl] = epk_v[b][1, sl]

        def issue_dscat(b):
            pltpu.async_copy(ones_v, deg_sh.at[dcol_v[b]], sem_d[b],
                             add=True)

        def wait_dscat(b):
            pltpu.make_async_copy(ones_v, deg_sh.at[dcol_v[b]],
                                  sem_d[b]).wait()

        def deg_iter(j, b, first):
            n = (b + 1) % 3
            p = (b + 2) % 3
            pltpu.make_async_copy(epk_hbm.at[s, j + 1], epk_v[n],
                                  sem_e).wait()
            prep_d(n)
            pltpu.async_copy(epk_hbm.at[s, j + 2], epk_v[p], sem_e)
            if not first:
                wait_dscat(p)
            issue_dscat(b)

        pltpu.sync_copy(epk_hbm.at[s, base], epk_v[0])
        prep_d(0)
        pltpu.async_copy(epk_hbm.at[s, base + 1], epk_v[1], sem_e)
        deg_iter(base, 0, True)
        deg_iter(base + 1, 1, False)

        def deg_triple(t, _):
            deg_iter(base + 3 * t + 2, 2, False)
            deg_iter(base + 3 * t + 3, 0, False)
            deg_iter(base + 3 * t + 4, 1, False)
            return 0
        lax.fori_loop(0, (NCC - 3) // 3, deg_triple, 0)
        deg_iter(base + NCC - 1, (NCC - 1) % 3, False)
        wait_dscat((NCC - 1) % 3)
        pltpu.make_async_copy(epk_hbm.at[s, base + NCC + 1],
                              epk_v[(NCC + 1) % 3], sem_e).wait()
        plsc.subcore_barrier()

        pltpu.sync_copy(deg_sh.at[pl.ds(nbase, NPS)], t_v)
        pltpu.sync_copy(t_v, pdeg_out.at[c, pl.ds(nbase, NPS)])

    return kdeg


def _make_sc_propagate(final: bool):
    mesh = plsc.VectorSubcoreMesh(core_axis_name="c", subcore_axis_name="s",
                                  num_cores=NC)
    if final:
        out_type = jax.ShapeDtypeStruct((N, D), jnp.float32)
    else:
        out_type = jax.ShapeDtypeStruct((NC * NP, HALF), jnp.float32)

    scratch = dict(
        epk_v=[pltpu.VMEM((3, CH), jnp.int32)] * 3,
        rowo_v=[pltpu.VMEM((CH,), jnp.int32)] * 3,
        col_v=[pltpu.VMEM((CH,), jnp.int32)] * 3,
        w_v=[pltpu.VMEM((CH,), jnp.float32)] * 3,
        rows_v=[pltpu.VMEM((CH, HALF), jnp.float32)] * 3,
        dis_v=pltpu.VMEM((NP,), jnp.float32),
        acc_sh=pltpu.VMEM_SHARED((NP, HALF), jnp.float32),
        sem_e=pltpu.SemaphoreType.DMA,
        sem_g=[pltpu.SemaphoreType.DMA] * 3,
        sem_s=[pltpu.SemaphoreType.DMA] * 3,
    )

    @functools.partial(
        pl.kernel, out_type=out_type, mesh=mesh, scratch_types=scratch,
        compiler_params=pltpu.CompilerParams(needs_layout_passes=False))
    def kprop(epk_hbm, hw_hbm, parts_hbm, h_out, *, epk_v, rowo_v, col_v,
              w_v, rows_v, dis_v, acc_sh, sem_e, sem_g, sem_s):
        c = lax.axis_index("c")
        s = lax.axis_index("s")
        nbase = s * NPS
        coff = jnp.full((L,), c * NP, jnp.int32)

        pltpu.sync_copy(parts_hbm.at[0], rows_v[0])
        pltpu.sync_copy(parts_hbm.at[1], rows_v[1])

        @plsc.parallel_loop(0, NP // HALF, unroll=2)
        def mk_dis(r):
            for g in range(HALF // L):
                sl = pl.ds(g * L, L)
                d = rows_v[0][r, sl] + rows_v[1][r, sl] + 1.0
                dis_v[pl.ds(r * HALF + g * L, L)] = _rsqrt16(d)

        def init_chunk(k, _):
            base = nbase + k * CH
            buf = rows_v[0]
            pltpu.sync_copy(hw_hbm.at[pl.ds(c * NP + base, CH)], buf)

            @plsc.parallel_loop(0, CH, unroll=4)
            def init_e(e):
                dvec = _splat(dis_v, base + e)
                for g in range(HALF // L):
                    sl = pl.ds(g * L, L)
                    buf[e, sl] = buf[e, sl] * dvec
            pltpu.sync_copy(buf, acc_sh.at[pl.ds(base, CH)])
            return 0
        lax.fori_loop(0, NNCH, init_chunk, 0)
        plsc.subcore_barrier()

        def prep(b):
            @plsc.parallel_loop(0, CH // L, unroll=5)
            def grp(g):
                sl = pl.ds(g * L, L)
                r16 = epk_v[b][0, sl]
                rowo_v[b][sl] = r16 + coff
                d16 = plsc.load_gather(dis_v, [r16])
                e16 = lax.bitcast_convert_type(epk_v[b][2, sl], jnp.float32)
                w_v[b][sl] = d16 * e16
                col_v[b][sl] = epk_v[b][1, sl]

        def issue_gather(b):
            pltpu.async_copy(hw_hbm.at[rowo_v[b]], rows_v[b], sem_g[b])

        def wait_gather(b):
            pltpu.make_async_copy(hw_hbm.at[rowo_v[b]], rows_v[b],
                                  sem_g[b]).wait()

        def issue_scatter(b):
            pltpu.async_copy(rows_v[b], acc_sh.at[col_v[b]], sem_s[b],
                             add=True)

        def wait_scatter(b):
            pltpu.make_async_copy(rows_v[b], acc_sh.at[col_v[b]],
                                  sem_s[b]).wait()

        def scale(b):
            @plsc.parallel_loop(0, CH, unroll=4)
            def scale_e(e):
                wvec = _splat(w_v[b], e)
                for g in range(HALF // L):
                    sl = pl.ds(g * L, L)
                    rows_v[b][e, sl] = rows_v[b][e, sl] * wvec

        def edge_iter(j, b, first):
            n = (b + 1) % 3
            p = (b + 2) % 3
            pltpu.make_async_copy(epk_hbm.at[s, j + 1], epk_v[n],
                                  sem_e).wait()
            prep(n)
            issue_gather(n)
            pltpu.async_copy(epk_hbm.at[s, j + 2], epk_v[p], sem_e)
            wait_gather(b)
            scale(b)
            if not first:
                wait_scatter(p)
            issue_scatter(b)

        pltpu.sync_copy(epk_hbm.at[s, 0], epk_v[0])
        prep(0)
        issue_gather(0)
        pltpu.async_copy(epk_hbm.at[s, 1], epk_v[1], sem_e)
        edge_iter(0, 0, True)
        edge_iter(1, 1, False)

        def edge_triple(t, _):
            edge_iter(3 * t + 2, 2, False)
            edge_iter(3 * t + 3, 0, False)
            edge_iter(3 * t + 4, 1, False)
            return 0
        lax.fori_loop(0, (NECH - 2) // 3, edge_triple, 0)
        wait_scatter(1)
        wait_gather(2)
        pltpu.make_async_copy(epk_hbm.at[s, NECH + 1],
                              epk_v[(NECH + 1) % 3], sem_e).wait()
        plsc.subcore_barrier()

        def flush_chunk(k, _):
            base = nbase + k * CH
            buf = rows_v[0]

            def do_flush():
                pltpu.sync_copy(acc_sh.at[pl.ds(base, CH)], buf)

                @plsc.parallel_loop(0, CH, unroll=4)
                def flush_e(e):
                    dvec = _splat(dis_v, base + e)
                    for g in range(HALF // L):
                        sl = pl.ds(g * L, L)
                        v = buf[e, sl] * dvec
                        if final:
                            v = 1.0 / (1.0 + jnp.exp(-v))
                        else:
                            v = jnp.maximum(v, 0.0)
                        buf[e, sl] = v
                if final:
                    pltpu.sync_copy(
                        buf, h_out.at[pl.ds(base, CH),
                                      pl.ds(c * HALF, HALF)])
                else:
                    pltpu.sync_copy(
                        buf, h_out.at[pl.ds(c * NP + base, CH)])

            if final:
                @pl.when(base < N)
                def _():
                    do_flush()
            else:
                do_flush()
            return 0
        lax.fori_loop(0, NNCH, flush_chunk, 0)

    return kprop


_sc_deg = _sc_degree()
_sc_prop_mid = _make_sc_propagate(False)
_sc_prop_final = _make_sc_propagate(True)


@jax.jit
def kernel(x, edge_index, edge_weight, W1, b1, W2, b2):
    row = edge_index[0].reshape(NS, NECH, CH)
    col = edge_index[1].reshape(NS, NECH, CH)
    ewb = lax.bitcast_convert_type(
        edge_weight.reshape(NS, NECH, CH), jnp.int32)
    epk = jnp.stack([row, col, ewb], axis=2)
    npad = 3
    rowp = jnp.zeros((NS, npad, CH), jnp.int32)
    colp = (N + (jnp.arange(NS * npad * CH, dtype=jnp.int32) % (NP - N))
            ).reshape(NS, npad, CH)
    ewpb = jnp.zeros((NS, npad, CH), jnp.int32)
    epk = jnp.concatenate(
        [epk, jnp.stack([rowp, colp, ewpb], axis=2)], axis=1)

    pdeg = _sc_deg(epk)
    parts = pdeg.reshape(NC, NP // HALF, HALF)
    xp = jnp.pad(x, ((0, NP - N), (0, 0)))
    hw1 = _linear_halves_from_full(xp, W1, b1).reshape(NC * NP, HALF)
    h1 = _sc_prop_mid(epk, hw1, parts).reshape(NC, NP, HALF)
    hw2 = _linear_halves_from_halves(h1, W2, b2).reshape(NC * NP, HALF)
    return _sc_prop_final(epk, hw2, parts)
